# baseline (speedup 1.0000x reference)
import functools

import numpy as np
import jax
import jax.numpy as jnp
from jax import lax
from jax.experimental import pallas as pl
from jax.experimental.pallas import tpu as pltpu
from jax.experimental.pallas import tpu_sc as plsc

N = 10000
E = 160000
T = 24
F = 8
H = 64
DT = 3600.0

NC = 2
NS = 16
NW = NC * NS
CH = 128
EPW = 5120
EPAD = EPW * NW
NCHUNK = EPW // CH
NPAD = 10112
ZROWS = NPAD // NS
OROWS = NPAD // NS

LSTM_BLK = 1000
DENSE_BLK = 2000


def _lstm_body(x_ref, wih_ref, whh_ref, b_ref, out_ref):
    wih = wih_ref[...]
    whh = whh_ref[...]
    b = b_ref[...]

    def step(t, carry):
        h, c = carry
        xt = x_ref[t]
        g = (jnp.dot(xt, wih, preferred_element_type=jnp.float32)
             + jnp.dot(h, whh, preferred_element_type=jnp.float32)
             + b).astype(jnp.bfloat16)
        sf = jax.nn.sigmoid(g[:, 0:2 * H])
        i = sf[:, 0:H]
        f = sf[:, H:2 * H]
        gg = jnp.tanh(g[:, 2 * H:3 * H])
        o = jax.nn.sigmoid(g[:, 3 * H:4 * H])
        c = f.astype(jnp.float32) * c + (i * gg).astype(jnp.float32)
        h = o * jnp.tanh(c.astype(jnp.bfloat16))
        return (h, c)

    zb = jnp.zeros((LSTM_BLK, H), jnp.bfloat16)
    z = jnp.zeros((LSTM_BLK, H), jnp.float32)
    h, _ = lax.fori_loop(0, T, step, (zb, z))
    out_ref[...] = h


def _lstm_encode(x3, wih_b, whh_b, b2d):
    grid = (N // LSTM_BLK,)
    return pl.pallas_call(
        _lstm_body,
        grid=grid,
        in_specs=[
            pl.BlockSpec((T, LSTM_BLK, F), lambda i: (0, i, 0)),
            pl.BlockSpec((F, 4 * H), lambda i: (0, 0)),
            pl.BlockSpec((H, 4 * H), lambda i: (0, 0)),
            pl.BlockSpec((1, 4 * H), lambda i: (0, 0)),
        ],
        out_specs=pl.BlockSpec((LSTM_BLK, H), lambda i: (i, 0)),
        out_shape=jax.ShapeDtypeStruct((N, H), jnp.bfloat16),
    )(x3, wih_b, whh_b, b2d)


NB = 4
NCHW = EPW // CH


def _agg_body(h_hbm, src_hbm, dst_hbm, k_hbm, x_hbm, out_hbm,
              src_all, dst_all, kk_all, xx_all,
              rows_l, stage_l, zbuf, agg_sh, gsem_l, ssem_l):
    c = lax.axis_index("c")
    s = lax.axis_index("s")
    wid = c * NS + s
    rows = rows_l
    stage = stage_l
    gsem = gsem_l
    ssem = ssem_l

    row0 = wid * NCHW
    pltpu.sync_copy(src_hbm.at[pl.ds(row0, NCHW)], src_all)
    pltpu.sync_copy(dst_hbm.at[pl.ds(row0, NCHW)], dst_all)
    pltpu.sync_copy(k_hbm.at[pl.ds(row0, NCHW)], kk_all)
    pltpu.sync_copy(x_hbm.at[pl.ds(row0, NCHW)], xx_all)

    for b in range(NB):
        pltpu.async_copy(h_hbm.at[src_all.at[b]], rows[b], gsem[b])

    def zrow(i, _):
        for j in range(H // 16):
            zbuf[i, pl.ds(j * 16, 16)] = jnp.zeros((16,), jnp.float32)
        return 0
    lax.fori_loop(0, ZROWS // 4, zrow, 0)
    for m in range(4):
        pltpu.sync_copy(zbuf,
                        agg_sh.at[pl.ds(s * ZROWS + m * (ZROWS // 4),
                                        ZROWS // 4)])
    plsc.subcore_barrier()

    MSK = jnp.int32(-65536)

    def scale(rv, sv, k):
        def grp(g, _):
            kk = kk_all[k, pl.ds(g * 16, 16)]
            xx = xx_all[k, pl.ds(g * 16, 16)]
            wv = (2.0 * DT) / (2.0 * kk * (1.0 - xx) + DT)
            for l in range(16):
                e = g * 16 + l
                spl = lax.broadcast(wv[l], (16,))
                for m in range(H // 32):
                    vi = rv[e, pl.ds(m * 16, 16)]
                    lo = lax.bitcast_convert_type(vi << 16, jnp.float32)
                    hi = lax.bitcast_convert_type(vi & MSK, jnp.float32)
                    sv[e, pl.ds(m * 32, 16)] = lo * spl
                    sv[e, pl.ds(m * 32 + 16, 16)] = hi * spl
            return 0
        lax.fori_loop(0, CH // 16, grp, 0)

    def group(g, _):
        for b in range(NB):
            k = g * NB + b
            pltpu.make_async_copy(h_hbm.at[src_all.at[k]],
                                  rows[b], gsem[b]).wait()
            @pl.when(k >= NB)
            def _():
                pltpu.make_async_copy(stage[b], agg_sh.at[dst_all.at[k]],
                                      ssem[b]).wait()
            scale(rows[b], stage[b], k)
            pltpu.async_copy(stage[b], agg_sh.at[dst_all.at[k]], ssem[b],
                             add=True)
            kn = k + NB
            @pl.when(kn < NCHW)
            def _():
                pltpu.async_copy(h_hbm.at[src_all.at[kn]], rows[b], gsem[b])
        return 0
    lax.fori_loop(0, NCHW // NB, group, 0)
    for b in range(NB):
        pltpu.make_async_copy(stage[b], agg_sh.at[dst_all.at[NCHW - NB + b]],
                              ssem[b]).wait()
    plsc.subcore_barrier()

    pltpu.sync_copy(agg_sh.at[pl.ds(s * OROWS, OROWS)],
                    out_hbm.at[c, pl.ds(s * OROWS, OROWS)])


def _sc_aggregate(h, src2, dst2, k2, x2):
    mesh = plsc.VectorSubcoreMesh(core_axis_name="c", subcore_axis_name="s",
                                  num_cores=NC, num_subcores=NS)
    fn = functools.partial(
        pl.kernel,
        out_type=jax.ShapeDtypeStruct((NC, NPAD, H), jnp.float32),
        mesh=mesh,
        compiler_params=pltpu.CompilerParams(use_tc_tiling_on_sc=False),
        scratch_types=[
            pltpu.VMEM((NCHW, CH), jnp.int32),
            pltpu.VMEM((NCHW, CH), jnp.int32),
            pltpu.VMEM((NCHW, CH), jnp.float32),
            pltpu.VMEM((NCHW, CH), jnp.float32),
            [pltpu.VMEM((CH, H // 2), jnp.int32) for _ in range(NB)],
            [pltpu.VMEM((CH, H), jnp.float32) for _ in range(NB)],
            pltpu.VMEM((ZROWS // 4, H), jnp.float32),
            pltpu.VMEM_SHARED((NPAD, H), jnp.float32),
            [pltpu.SemaphoreType.DMA for _ in range(NB)],
            [pltpu.SemaphoreType.DMA for _ in range(NB)],
        ],
    )(_agg_body)
    return fn(h, src2, dst2, k2, x2)


def _dense_body(p_ref, h_ref, wm_ref, ws_ref, bg_ref, out_ref):
    agg = p_ref[0] + p_ref[1]
    h = h_ref[...].astype(jnp.float32)
    upd = (jnp.dot(agg, wm_ref[...], preferred_element_type=jnp.float32)
           + jnp.dot(h, ws_ref[...], preferred_element_type=jnp.float32)
           + bg_ref[...])
    out_ref[...] = (h + jax.nn.gelu(upd)).astype(jnp.bfloat16)


def _dense_layer(p, h, wm_b, ws_b, bg2d):
    grid = (N // DENSE_BLK,)
    return pl.pallas_call(
        _dense_body,
        grid=grid,
        in_specs=[
            pl.BlockSpec((NC, DENSE_BLK, H), lambda i: (0, i, 0)),
            pl.BlockSpec((DENSE_BLK, H), lambda i: (i, 0)),
            pl.BlockSpec((H, H), lambda i: (0, 0)),
            pl.BlockSpec((H, H), lambda i: (0, 0)),
            pl.BlockSpec((1, H), lambda i: (0, 0)),
        ],
        out_specs=pl.BlockSpec((DENSE_BLK, H), lambda i: (i, 0)),
        out_shape=jax.ShapeDtypeStruct((N, H), jnp.bfloat16),
    )(p, h, wm_b, ws_b, bg2d)


def _dense_head_body(p_ref, h_ref, wm_ref, ws_ref, bg_ref,
                     w1_ref, b1_ref, w2_ref, b2_ref, out_ref):
    agg = p_ref[0] + p_ref[1]
    h = h_ref[...].astype(jnp.float32)
    upd = (jnp.dot(agg, wm_ref[...], preferred_element_type=jnp.float32)
           + jnp.dot(h, ws_ref[...], preferred_element_type=jnp.float32)
           + bg_ref[...])
    h1 = h + jax.nn.gelu(upd)
    z = (jnp.dot(h1, w1_ref[...],
                 preferred_element_type=jnp.float32) + b1_ref[...])
    y = (jnp.dot(jax.nn.gelu(z), w2_ref[...],
                 preferred_element_type=jnp.float32) + b2_ref[...])
    out_ref[...] = y


def _dense_head_layer(p, h, wm_b, ws_b, bg2d, w1_b, b12d, w2_b, b22d):
    grid = (N // DENSE_BLK,)
    return pl.pallas_call(
        _dense_head_body,
        grid=grid,
        in_specs=[
            pl.BlockSpec((NC, DENSE_BLK, H), lambda i: (0, i, 0)),
            pl.BlockSpec((DENSE_BLK, H), lambda i: (i, 0)),
            pl.BlockSpec((H, H), lambda i: (0, 0)),
            pl.BlockSpec((H, H), lambda i: (0, 0)),
            pl.BlockSpec((1, H), lambda i: (0, 0)),
            pl.BlockSpec((H, H), lambda i: (0, 0)),
            pl.BlockSpec((1, H), lambda i: (0, 0)),
            pl.BlockSpec((H, 1), lambda i: (0, 0)),
            pl.BlockSpec((1, 1), lambda i: (0, 0)),
        ],
        out_specs=pl.BlockSpec((DENSE_BLK, 1), lambda i: (i, 0)),
        out_shape=jax.ShapeDtypeStruct((N, 1), jnp.float32),
    )(p, h, wm_b, ws_b, bg2d, w1_b, b12d, w2_b, b22d)


def kernel(x, edge_index, W_ih, W_hh, b_lstm, K_e, X_e, Ws, Wm, bg,
           W1, b1, W2, b2):
    x3 = x.reshape(N, T, F).transpose(1, 0, 2).astype(jnp.bfloat16)
    src = edge_index[0].astype(jnp.int32)
    dst = edge_index[1].astype(jnp.int32)
    npad = EPAD - E
    srcp = jnp.concatenate([src, jnp.zeros((npad,), jnp.int32)]
                           ).reshape(EPAD // CH, CH)
    dstp = jnp.concatenate([dst, jnp.full((npad,), N, jnp.int32)]
                           ).reshape(EPAD // CH, CH)
    kp = jnp.concatenate([K_e, jnp.full((npad,), DT, jnp.float32)]
                         ).reshape(EPAD // CH, CH)
    xp = jnp.concatenate([X_e, jnp.zeros((npad,), jnp.float32)]
                         ).reshape(EPAD // CH, CH)

    perm = np.concatenate([np.concatenate([np.arange(m * 32, (m + 1) * 32, 2),
                                           np.arange(m * 32 + 1, (m + 1) * 32, 2)])
                           for m in range(H // 32)])
    h = _lstm_encode(x3,
                     W_ih.astype(jnp.bfloat16),
                     W_hh.astype(jnp.bfloat16),
                     b_lstm.reshape(1, 4 * H))

    def bc(hb):
        return lax.bitcast_convert_type(hb.reshape(N, H // 2, 2), jnp.int32)

    p = _sc_aggregate(bc(h), srcp, dstp, kp, xp)
    h = _dense_layer(p, h, Wm[0][perm, :], Ws[0], bg[0].reshape(1, H))
    p = _sc_aggregate(bc(h), srcp, dstp, kp, xp)
    y = _dense_head_layer(p, h, Wm[1][perm, :], Ws[1], bg[1].reshape(1, H),
                          W1, b1.reshape(1, H), W2, b2.reshape(1, 1))
    return y.reshape(1, N)


# speedup vs baseline: 1.0602x; 1.0602x over previous
import functools

import numpy as np
import jax
import jax.numpy as jnp
from jax import lax
from jax.experimental import pallas as pl
from jax.experimental.pallas import tpu as pltpu
from jax.experimental.pallas import tpu_sc as plsc

N = 10000
E = 160000
T = 24
F = 8
H = 64
DT = 3600.0

NC = 2
NS = 16
NW = NC * NS
CH = 128
EPW = 5120
EPAD = EPW * NW
NCHUNK = EPW // CH
NPAD = 10112
ZROWS = NPAD // NS
OROWS = NPAD // NS

LSTM_BLK = 1000
DENSE_BLK = 2000


def _lstm_body(x_ref, wih_ref, whh_ref, b_ref, out_ref):
    wih = wih_ref[...]
    whh = whh_ref[...]
    b = b_ref[...]

    def step(t, carry):
        h, c = carry
        xt = x_ref[t].astype(jnp.bfloat16)
        g = (jnp.dot(xt, wih, preferred_element_type=jnp.float32)
             + jnp.dot(h.astype(jnp.bfloat16), whh,
                       preferred_element_type=jnp.float32)
             + b)
        sf = 0.5 + 0.5 * jnp.tanh(0.5 * g[:, 0:2 * H])
        i = sf[:, 0:H]
        f = sf[:, H:2 * H]
        gg = jnp.tanh(g[:, 2 * H:3 * H])
        o = 0.5 + 0.5 * jnp.tanh(0.5 * g[:, 3 * H:4 * H])
        c = f * c + i * gg
        h = o * jnp.tanh(c)
        return (h, c)

    z = jnp.zeros((LSTM_BLK, H), jnp.float32)
    h, _ = lax.fori_loop(0, T, step, (z, z))
    out_ref[...] = h.astype(jnp.bfloat16)


def _lstm_encode(x3, wih_b, whh_b, b2d):
    grid = (N // LSTM_BLK,)
    return pl.pallas_call(
        _lstm_body,
        grid=grid,
        in_specs=[
            pl.BlockSpec((T, LSTM_BLK, F), lambda i: (0, i, 0)),
            pl.BlockSpec((F, 4 * H), lambda i: (0, 0)),
            pl.BlockSpec((H, 4 * H), lambda i: (0, 0)),
            pl.BlockSpec((1, 4 * H), lambda i: (0, 0)),
        ],
        out_specs=pl.BlockSpec((LSTM_BLK, H), lambda i: (i, 0)),
        out_shape=jax.ShapeDtypeStruct((N, H), jnp.bfloat16),
    )(x3, wih_b, whh_b, b2d)


NB = 4
NCHW = EPW // CH


def _agg_body(h_hbm, src_hbm, dst_hbm, k_hbm, x_hbm, out_hbm,
              src_all, dst_all, kk_all, xx_all,
              rows_l, stage_l, zbuf, agg_sh, gsem_l, ssem_l):
    c = lax.axis_index("c")
    s = lax.axis_index("s")
    wid = c * NS + s
    rows = rows_l
    stage = stage_l
    gsem = gsem_l
    ssem = ssem_l

    row0 = wid * NCHW
    pltpu.sync_copy(src_hbm.at[pl.ds(row0, NCHW)], src_all)
    pltpu.sync_copy(dst_hbm.at[pl.ds(row0, NCHW)], dst_all)
    pltpu.sync_copy(k_hbm.at[pl.ds(row0, NCHW)], kk_all)
    pltpu.sync_copy(x_hbm.at[pl.ds(row0, NCHW)], xx_all)

    for b in range(NB):
        pltpu.async_copy(h_hbm.at[src_all.at[b]], rows[b], gsem[b])

    def zrow(i, _):
        for j in range(H // 16):
            zbuf[i, pl.ds(j * 16, 16)] = jnp.zeros((16,), jnp.float32)
        return 0
    lax.fori_loop(0, ZROWS // 4, zrow, 0)
    for m in range(4):
        pltpu.sync_copy(zbuf,
                        agg_sh.at[pl.ds(s * ZROWS + m * (ZROWS // 4),
                                        ZROWS // 4)])
    plsc.subcore_barrier()

    MSK = jnp.int32(-65536)

    def scale(rv, sv, k):
        def grp(g, _):
            kk = kk_all[k, pl.ds(g * 16, 16)]
            xx = xx_all[k, pl.ds(g * 16, 16)]
            wv = (2.0 * DT) / (2.0 * kk * (1.0 - xx) + DT)
            for l in range(16):
                e = g * 16 + l
                spl = lax.broadcast(wv[l], (16,))
                for m in range(H // 32):
                    vi = rv[e, pl.ds(m * 16, 16)]
                    lo = lax.bitcast_convert_type(vi << 16, jnp.float32)
                    hi = lax.bitcast_convert_type(vi & MSK, jnp.float32)
                    sv[e, pl.ds(m * 32, 16)] = lo * spl
                    sv[e, pl.ds(m * 32 + 16, 16)] = hi * spl
            return 0
        lax.fori_loop(0, CH // 16, grp, 0)

    def group(g, _):
        for b in range(NB):
            k = g * NB + b
            pltpu.make_async_copy(h_hbm.at[src_all.at[k]],
                                  rows[b], gsem[b]).wait()
            @pl.when(k >= NB)
            def _():
                pltpu.make_async_copy(stage[b], agg_sh.at[dst_all.at[k]],
                                      ssem[b]).wait()
            scale(rows[b], stage[b], k)
            pltpu.async_copy(stage[b], agg_sh.at[dst_all.at[k]], ssem[b],
                             add=True)
            kn = k + NB
            @pl.when(kn < NCHW)
            def _():
                pltpu.async_copy(h_hbm.at[src_all.at[kn]], rows[b], gsem[b])
        return 0
    lax.fori_loop(0, NCHW // NB, group, 0)
    for b in range(NB):
        pltpu.make_async_copy(stage[b], agg_sh.at[dst_all.at[NCHW - NB + b]],
                              ssem[b]).wait()
    plsc.subcore_barrier()

    pltpu.sync_copy(agg_sh.at[pl.ds(s * OROWS, OROWS)],
                    out_hbm.at[c, pl.ds(s * OROWS, OROWS)])


def _sc_aggregate(h, src2, dst2, k2, x2):
    mesh = plsc.VectorSubcoreMesh(core_axis_name="c", subcore_axis_name="s",
                                  num_cores=NC, num_subcores=NS)
    fn = functools.partial(
        pl.kernel,
        out_type=jax.ShapeDtypeStruct((NC, NPAD, H), jnp.float32),
        mesh=mesh,
        compiler_params=pltpu.CompilerParams(use_tc_tiling_on_sc=False),
        scratch_types=[
            pltpu.VMEM((NCHW, CH), jnp.int32),
            pltpu.VMEM((NCHW, CH), jnp.int32),
            pltpu.VMEM((NCHW, CH), jnp.float32),
            pltpu.VMEM((NCHW, CH), jnp.float32),
            [pltpu.VMEM((CH, H // 2), jnp.int32) for _ in range(NB)],
            [pltpu.VMEM((CH, H), jnp.float32) for _ in range(NB)],
            pltpu.VMEM((ZROWS // 4, H), jnp.float32),
            pltpu.VMEM_SHARED((NPAD, H), jnp.float32),
            [pltpu.SemaphoreType.DMA for _ in range(NB)],
            [pltpu.SemaphoreType.DMA for _ in range(NB)],
        ],
    )(_agg_body)
    return fn(h, src2, dst2, k2, x2)


def _dense_body(p_ref, h_ref, wm_ref, ws_ref, bg_ref, out_ref):
    agg = p_ref[0] + p_ref[1]
    h = h_ref[...].astype(jnp.float32)
    upd = (jnp.dot(agg, wm_ref[...], preferred_element_type=jnp.float32)
           + jnp.dot(h, ws_ref[...], preferred_element_type=jnp.float32)
           + bg_ref[...])
    out_ref[...] = (h + jax.nn.gelu(upd)).astype(jnp.bfloat16)


def _dense_layer(p, h, wm_b, ws_b, bg2d):
    grid = (N // DENSE_BLK,)
    return pl.pallas_call(
        _dense_body,
        grid=grid,
        in_specs=[
            pl.BlockSpec((NC, DENSE_BLK, H), lambda i: (0, i, 0)),
            pl.BlockSpec((DENSE_BLK, H), lambda i: (i, 0)),
            pl.BlockSpec((H, H), lambda i: (0, 0)),
            pl.BlockSpec((H, H), lambda i: (0, 0)),
            pl.BlockSpec((1, H), lambda i: (0, 0)),
        ],
        out_specs=pl.BlockSpec((DENSE_BLK, H), lambda i: (i, 0)),
        out_shape=jax.ShapeDtypeStruct((N, H), jnp.bfloat16),
    )(p, h, wm_b, ws_b, bg2d)


def _dense_head_body(p_ref, h_ref, wm_ref, ws_ref, bg_ref,
                     w1_ref, b1_ref, w2_ref, b2_ref, out_ref):
    agg = p_ref[0] + p_ref[1]
    h = h_ref[...].astype(jnp.float32)
    upd = (jnp.dot(agg, wm_ref[...], preferred_element_type=jnp.float32)
           + jnp.dot(h, ws_ref[...], preferred_element_type=jnp.float32)
           + bg_ref[...])
    h1 = h + jax.nn.gelu(upd)
    z = (jnp.dot(h1, w1_ref[...],
                 preferred_element_type=jnp.float32) + b1_ref[...])
    y = (jnp.dot(jax.nn.gelu(z), w2_ref[...],
                 preferred_element_type=jnp.float32) + b2_ref[...])
    out_ref[...] = y


def _dense_head_layer(p, h, wm_b, ws_b, bg2d, w1_b, b12d, w2_b, b22d):
    grid = (N // DENSE_BLK,)
    return pl.pallas_call(
        _dense_head_body,
        grid=grid,
        in_specs=[
            pl.BlockSpec((NC, DENSE_BLK, H), lambda i: (0, i, 0)),
            pl.BlockSpec((DENSE_BLK, H), lambda i: (i, 0)),
            pl.BlockSpec((H, H), lambda i: (0, 0)),
            pl.BlockSpec((H, H), lambda i: (0, 0)),
            pl.BlockSpec((1, H), lambda i: (0, 0)),
            pl.BlockSpec((H, H), lambda i: (0, 0)),
            pl.BlockSpec((1, H), lambda i: (0, 0)),
            pl.BlockSpec((H, 1), lambda i: (0, 0)),
            pl.BlockSpec((1, 1), lambda i: (0, 0)),
        ],
        out_specs=pl.BlockSpec((DENSE_BLK, 1), lambda i: (i, 0)),
        out_shape=jax.ShapeDtypeStruct((N, 1), jnp.float32),
    )(p, h, wm_b, ws_b, bg2d, w1_b, b12d, w2_b, b22d)


def kernel(x, edge_index, W_ih, W_hh, b_lstm, K_e, X_e, Ws, Wm, bg,
           W1, b1, W2, b2):
    x3 = x.reshape(N, T, F).transpose(1, 0, 2)
    src = edge_index[0].astype(jnp.int32)
    dst = edge_index[1].astype(jnp.int32)
    npad = EPAD - E
    srcp = jnp.concatenate([src, jnp.zeros((npad,), jnp.int32)]
                           ).reshape(EPAD // CH, CH)
    dstp = jnp.concatenate([dst, jnp.full((npad,), N, jnp.int32)]
                           ).reshape(EPAD // CH, CH)
    kp = jnp.concatenate([K_e, jnp.full((npad,), DT, jnp.float32)]
                         ).reshape(EPAD // CH, CH)
    xp = jnp.concatenate([X_e, jnp.zeros((npad,), jnp.float32)]
                         ).reshape(EPAD // CH, CH)

    perm = np.concatenate([np.concatenate([np.arange(m * 32, (m + 1) * 32, 2),
                                           np.arange(m * 32 + 1, (m + 1) * 32, 2)])
                           for m in range(H // 32)])
    h = _lstm_encode(x3,
                     W_ih.astype(jnp.bfloat16),
                     W_hh.astype(jnp.bfloat16),
                     b_lstm.reshape(1, 4 * H))

    def bc(hb):
        return lax.bitcast_convert_type(hb.reshape(N, H // 2, 2), jnp.int32)

    p = _sc_aggregate(bc(h), srcp, dstp, kp, xp)
    h = _dense_layer(p, h, Wm[0][perm, :], Ws[0], bg[0].reshape(1, H))
    p = _sc_aggregate(bc(h), srcp, dstp, kp, xp)
    y = _dense_head_layer(p, h, Wm[1][perm, :], Ws[1], bg[1].reshape(1, H),
                          W1, b1.reshape(1, H), W2, b2.reshape(1, 1))
    return y.reshape(1, N)


# speedup vs baseline: 1.1043x; 1.0416x over previous
import functools

import numpy as np
import jax
import jax.numpy as jnp
from jax import lax
from jax.experimental import pallas as pl
from jax.experimental.pallas import tpu as pltpu
from jax.experimental.pallas import tpu_sc as plsc

N = 10000
E = 160000
T = 24
F = 8
H = 64
DT = 3600.0

NC = 2
NS = 16
NW = NC * NS
CH = 128
EPW = 5120
EPAD = EPW * NW
NCHUNK = EPW // CH
NPAD = 10112
ZROWS = NPAD // NS
OROWS = NPAD // NS

LSTM_BLK = 2000
DENSE_BLK = 2000


def _lstm_body(x_ref, wih_ref, whh_ref, b_ref, out_ref):
    wih = wih_ref[...]
    whh = whh_ref[...]
    b = b_ref[...]

    def step(t, carry):
        h, c = carry
        xt = x_ref[t].astype(jnp.bfloat16)
        g = (jnp.dot(xt, wih, preferred_element_type=jnp.float32)
             + jnp.dot(h.astype(jnp.bfloat16), whh,
                       preferred_element_type=jnp.float32)
             + b)
        sf = 0.5 + 0.5 * jnp.tanh(0.5 * g[:, 0:2 * H])
        i = sf[:, 0:H]
        f = sf[:, H:2 * H]
        gg = jnp.tanh(g[:, 2 * H:3 * H])
        o = 0.5 + 0.5 * jnp.tanh(0.5 * g[:, 3 * H:4 * H])
        c = f * c + i * gg
        h = o * jnp.tanh(c)
        return (h, c)

    z = jnp.zeros((LSTM_BLK, H), jnp.float32)
    h, _ = lax.fori_loop(0, T, step, (z, z))
    out_ref[...] = h.astype(jnp.bfloat16)


def _lstm_encode(x3, wih_b, whh_b, b2d):
    grid = (N // LSTM_BLK,)
    return pl.pallas_call(
        _lstm_body,
        grid=grid,
        in_specs=[
            pl.BlockSpec((T, LSTM_BLK, F), lambda i: (0, i, 0)),
            pl.BlockSpec((F, 4 * H), lambda i: (0, 0)),
            pl.BlockSpec((H, 4 * H), lambda i: (0, 0)),
            pl.BlockSpec((1, 4 * H), lambda i: (0, 0)),
        ],
        out_specs=pl.BlockSpec((LSTM_BLK, H), lambda i: (i, 0)),
        out_shape=jax.ShapeDtypeStruct((N, H), jnp.bfloat16),
    )(x3, wih_b, whh_b, b2d)


NB = 4
NCHW = EPW // CH


def _agg_body(h_hbm, src_hbm, dst_hbm, k_hbm, x_hbm, out_hbm,
              src_all, dst_all, kk_all, xx_all,
              rows_l, stage_l, zbuf, agg_sh, gsem_l, ssem_l):
    c = lax.axis_index("c")
    s = lax.axis_index("s")
    wid = c * NS + s
    rows = rows_l
    stage = stage_l
    gsem = gsem_l
    ssem = ssem_l

    row0 = wid * NCHW
    pltpu.sync_copy(src_hbm.at[pl.ds(row0, NCHW)], src_all)
    pltpu.sync_copy(dst_hbm.at[pl.ds(row0, NCHW)], dst_all)
    pltpu.sync_copy(k_hbm.at[pl.ds(row0, NCHW)], kk_all)
    pltpu.sync_copy(x_hbm.at[pl.ds(row0, NCHW)], xx_all)

    for b in range(NB):
        pltpu.async_copy(h_hbm.at[src_all.at[b]], rows[b], gsem[b])

    def zrow(i, _):
        for j in range(H // 16):
            zbuf[i, pl.ds(j * 16, 16)] = jnp.zeros((16,), jnp.float32)
        return 0
    lax.fori_loop(0, ZROWS // 4, zrow, 0)
    for m in range(4):
        pltpu.sync_copy(zbuf,
                        agg_sh.at[pl.ds(s * ZROWS + m * (ZROWS // 4),
                                        ZROWS // 4)])
    plsc.subcore_barrier()

    MSK = jnp.int32(-65536)

    def scale(rv, sv, k):
        def grp(g, _):
            kk = kk_all[k, pl.ds(g * 16, 16)]
            xx = xx_all[k, pl.ds(g * 16, 16)]
            wv = (2.0 * DT) / (2.0 * kk * (1.0 - xx) + DT)
            for l in range(16):
                e = g * 16 + l
                spl = lax.broadcast(wv[l], (16,))
                for m in range(H // 32):
                    vi = rv[e, pl.ds(m * 16, 16)]
                    lo = lax.bitcast_convert_type(vi << 16, jnp.float32)
                    hi = lax.bitcast_convert_type(vi & MSK, jnp.float32)
                    sv[e, pl.ds(m * 32, 16)] = lo * spl
                    sv[e, pl.ds(m * 32 + 16, 16)] = hi * spl
            return 0
        lax.fori_loop(0, CH // 16, grp, 0)

    def group(g, _):
        for b in range(NB):
            k = g * NB + b
            pltpu.make_async_copy(h_hbm.at[src_all.at[k]],
                                  rows[b], gsem[b]).wait()
            @pl.when(k >= NB)
            def _():
                pltpu.make_async_copy(stage[b], agg_sh.at[dst_all.at[k]],
                                      ssem[b]).wait()
            scale(rows[b], stage[b], k)
            pltpu.async_copy(stage[b], agg_sh.at[dst_all.at[k]], ssem[b],
                             add=True)
            kn = k + NB
            @pl.when(kn < NCHW)
            def _():
                pltpu.async_copy(h_hbm.at[src_all.at[kn]], rows[b], gsem[b])
        return 0
    lax.fori_loop(0, NCHW // NB, group, 0)
    for b in range(NB):
        pltpu.make_async_copy(stage[b], agg_sh.at[dst_all.at[NCHW - NB + b]],
                              ssem[b]).wait()
    plsc.subcore_barrier()

    pltpu.sync_copy(agg_sh.at[pl.ds(s * OROWS, OROWS)],
                    out_hbm.at[c, pl.ds(s * OROWS, OROWS)])


def _sc_aggregate(h, src2, dst2, k2, x2):
    mesh = plsc.VectorSubcoreMesh(core_axis_name="c", subcore_axis_name="s",
                                  num_cores=NC, num_subcores=NS)
    fn = functools.partial(
        pl.kernel,
        out_type=jax.ShapeDtypeStruct((NC, NPAD, H), jnp.float32),
        mesh=mesh,
        compiler_params=pltpu.CompilerParams(use_tc_tiling_on_sc=False),
        scratch_types=[
            pltpu.VMEM((NCHW, CH), jnp.int32),
            pltpu.VMEM((NCHW, CH), jnp.int32),
            pltpu.VMEM((NCHW, CH), jnp.float32),
            pltpu.VMEM((NCHW, CH), jnp.float32),
            [pltpu.VMEM((CH, H // 2), jnp.int32) for _ in range(NB)],
            [pltpu.VMEM((CH, H), jnp.float32) for _ in range(NB)],
            pltpu.VMEM((ZROWS // 4, H), jnp.float32),
            pltpu.VMEM_SHARED((NPAD, H), jnp.float32),
            [pltpu.SemaphoreType.DMA for _ in range(NB)],
            [pltpu.SemaphoreType.DMA for _ in range(NB)],
        ],
    )(_agg_body)
    return fn(h, src2, dst2, k2, x2)


def _dense_body(p_ref, h_ref, wm_ref, ws_ref, bg_ref, out_ref):
    agg = p_ref[0] + p_ref[1]
    h = h_ref[...].astype(jnp.float32)
    upd = (jnp.dot(agg, wm_ref[...], preferred_element_type=jnp.float32)
           + jnp.dot(h, ws_ref[...], preferred_element_type=jnp.float32)
           + bg_ref[...])
    out_ref[...] = (h + jax.nn.gelu(upd)).astype(jnp.bfloat16)


def _dense_layer(p, h, wm_b, ws_b, bg2d):
    grid = (N // DENSE_BLK,)
    return pl.pallas_call(
        _dense_body,
        grid=grid,
        in_specs=[
            pl.BlockSpec((NC, DENSE_BLK, H), lambda i: (0, i, 0)),
            pl.BlockSpec((DENSE_BLK, H), lambda i: (i, 0)),
            pl.BlockSpec((H, H), lambda i: (0, 0)),
            pl.BlockSpec((H, H), lambda i: (0, 0)),
            pl.BlockSpec((1, H), lambda i: (0, 0)),
        ],
        out_specs=pl.BlockSpec((DENSE_BLK, H), lambda i: (i, 0)),
        out_shape=jax.ShapeDtypeStruct((N, H), jnp.bfloat16),
    )(p, h, wm_b, ws_b, bg2d)


def _dense_head_body(p_ref, h_ref, wm_ref, ws_ref, bg_ref,
                     w1_ref, b1_ref, w2_ref, b2_ref, out_ref):
    agg = p_ref[0] + p_ref[1]
    h = h_ref[...].astype(jnp.float32)
    upd = (jnp.dot(agg, wm_ref[...], preferred_element_type=jnp.float32)
           + jnp.dot(h, ws_ref[...], preferred_element_type=jnp.float32)
           + bg_ref[...])
    h1 = h + jax.nn.gelu(upd)
    z = (jnp.dot(h1, w1_ref[...],
                 preferred_element_type=jnp.float32) + b1_ref[...])
    y = (jnp.dot(jax.nn.gelu(z), w2_ref[...],
                 preferred_element_type=jnp.float32) + b2_ref[...])
    out_ref[...] = y


def _dense_head_layer(p, h, wm_b, ws_b, bg2d, w1_b, b12d, w2_b, b22d):
    grid = (N // DENSE_BLK,)
    return pl.pallas_call(
        _dense_head_body,
        grid=grid,
        in_specs=[
            pl.BlockSpec((NC, DENSE_BLK, H), lambda i: (0, i, 0)),
            pl.BlockSpec((DENSE_BLK, H), lambda i: (i, 0)),
            pl.BlockSpec((H, H), lambda i: (0, 0)),
            pl.BlockSpec((H, H), lambda i: (0, 0)),
            pl.BlockSpec((1, H), lambda i: (0, 0)),
            pl.BlockSpec((H, H), lambda i: (0, 0)),
            pl.BlockSpec((1, H), lambda i: (0, 0)),
            pl.BlockSpec((H, 1), lambda i: (0, 0)),
            pl.BlockSpec((1, 1), lambda i: (0, 0)),
        ],
        out_specs=pl.BlockSpec((DENSE_BLK, 1), lambda i: (i, 0)),
        out_shape=jax.ShapeDtypeStruct((N, 1), jnp.float32),
    )(p, h, wm_b, ws_b, bg2d, w1_b, b12d, w2_b, b22d)


def kernel(x, edge_index, W_ih, W_hh, b_lstm, K_e, X_e, Ws, Wm, bg,
           W1, b1, W2, b2):
    x3 = x.reshape(N, T, F).transpose(1, 0, 2)
    src = edge_index[0].astype(jnp.int32)
    dst = edge_index[1].astype(jnp.int32)
    npad = EPAD - E
    srcp = jnp.concatenate([src, jnp.zeros((npad,), jnp.int32)]
                           ).reshape(EPAD // CH, CH)
    dstp = jnp.concatenate([dst, jnp.full((npad,), N, jnp.int32)]
                           ).reshape(EPAD // CH, CH)
    kp = jnp.concatenate([K_e, jnp.full((npad,), DT, jnp.float32)]
                         ).reshape(EPAD // CH, CH)
    xp = jnp.concatenate([X_e, jnp.zeros((npad,), jnp.float32)]
                         ).reshape(EPAD // CH, CH)

    perm = np.concatenate([np.concatenate([np.arange(m * 32, (m + 1) * 32, 2),
                                           np.arange(m * 32 + 1, (m + 1) * 32, 2)])
                           for m in range(H // 32)])
    h = _lstm_encode(x3,
                     W_ih.astype(jnp.bfloat16),
                     W_hh.astype(jnp.bfloat16),
                     b_lstm.reshape(1, 4 * H))

    def bc(hb):
        return lax.bitcast_convert_type(hb.reshape(N, H // 2, 2), jnp.int32)

    p = _sc_aggregate(bc(h), srcp, dstp, kp, xp)
    h = _dense_layer(p, h, Wm[0][perm, :], Ws[0], bg[0].reshape(1, H))
    p = _sc_aggregate(bc(h), srcp, dstp, kp, xp)
    y = _dense_head_layer(p, h, Wm[1][perm, :], Ws[1], bg[1].reshape(1, H),
                          W1, b1.reshape(1, H), W2, b2.reshape(1, 1))
    return y.reshape(1, N)
